# initial kernel scaffold (unmeasured)
import jax
import jax.numpy as jnp
from jax import lax
from jax.experimental import pallas as pl
from jax.experimental.pallas import tpu as pltpu

N_DEV = 32

_sem_signal = getattr(pl, "semaphore_signal", None) or pltpu.semaphore_signal
_sem_wait = getattr(pl, "semaphore_wait", None) or pltpu.semaphore_wait
_CompilerParams = getattr(pltpu, "CompilerParams", None) or pltpu.TPUCompilerParams


def kernel(x, w_mat):
    m, _ = x.shape
    _, n = w_mat.shape
    ch = m // N_DEV

    def body(x_ref, w_ref, out_ref, recv_ref, send_sems, recv_sems, credit_sems):
        my = lax.axis_index("i")
        left = (my - 1) % N_DEV
        right = (my + 1) % N_DEV

        barrier_sem = pltpu.get_barrier_semaphore()
        for nbr in (left, right):
            _sem_signal(barrier_sem, inc=1, device_id=(nbr,),
                        device_id_type=pl.DeviceIdType.MESH)
        _sem_wait(barrier_sem, 2)

        out_ref[...] = lax.dot_general(
            x_ref[...], w_ref[...],
            dimension_numbers=(((1,), (0,)), ((), ())),
            preferred_element_type=jnp.float32,
        )

        def hop(g, sa, do_credit_wait):
            slot = lax.rem(g, 2)

            @pl.when(do_credit_wait)
            def _():
                _sem_wait(credit_sems.at[slot], 1)

            rdma = pltpu.make_async_remote_copy(
                src_ref=out_ref.at[pl.ds(sa * ch, ch), :],
                dst_ref=recv_ref.at[slot],
                send_sem=send_sems.at[slot],
                recv_sem=recv_sems.at[slot],
                device_id=(right,),
                device_id_type=pl.DeviceIdType.MESH,
            )
            rdma.start()
            rdma.wait()
            return slot

        def rs_body(s, _):
            sa = (my - s) % N_DEV
            ra = (my - s - 1) % N_DEV
            slot = hop(s, sa, s >= 2)
            out_ref[pl.ds(ra * ch, ch), :] += recv_ref[slot]
            _sem_signal(credit_sems.at[slot], inc=1, device_id=(left,),
                        device_id_type=pl.DeviceIdType.MESH)
            return 0

        lax.fori_loop(0, N_DEV - 1, rs_body, 0)

        def ag_body(t, _):
            g = t + (N_DEV - 1)
            sa = (my + 1 - t) % N_DEV
            ra = (my - t) % N_DEV
            slot = hop(g, sa, g >= 2)
            out_ref[pl.ds(ra * ch, ch), :] = recv_ref[slot]
            _sem_signal(credit_sems.at[slot], inc=1, device_id=(left,),
                        device_id_type=pl.DeviceIdType.MESH)
            return 0

        lax.fori_loop(0, N_DEV - 1, ag_body, 0)

        _sem_wait(credit_sems.at[0], 1)
        _sem_wait(credit_sems.at[1], 1)

        y = jnp.maximum(out_ref[...], 0.0)
        amax = jnp.max(y)
        scale = amax / 127.0
        q = jnp.clip(jnp.round(y / scale), -127.0, 127.0)
        out_ref[...] = q * scale

    return pl.pallas_call(
        body,
        out_shape=jax.ShapeDtypeStruct((m, n), jnp.float32),
        in_specs=[
            pl.BlockSpec(memory_space=pltpu.VMEM),
            pl.BlockSpec(memory_space=pltpu.VMEM),
        ],
        out_specs=pl.BlockSpec(memory_space=pltpu.VMEM),
        scratch_shapes=[
            pltpu.VMEM((2, ch, n), jnp.float32),
            pltpu.SemaphoreType.DMA((2,)),
            pltpu.SemaphoreType.DMA((2,)),
            pltpu.SemaphoreType.REGULAR((2,)),
        ],
        compiler_params=_CompilerParams(collective_id=0),
    )(x, w_mat)


# baseline (device time: 882065 ns/iter reference)
import jax
import jax.numpy as jnp
from jax import lax
from jax.experimental import pallas as pl
from jax.experimental.pallas import tpu as pltpu

N_DEV = 32

_sem_signal = getattr(pl, "semaphore_signal", None) or pltpu.semaphore_signal
_sem_wait = getattr(pl, "semaphore_wait", None) or pltpu.semaphore_wait
_CompilerParams = getattr(pltpu, "CompilerParams", None) or pltpu.TPUCompilerParams


def kernel(x, w_mat):
    m, _ = x.shape
    _, n = w_mat.shape
    ch = m // N_DEV

    def body(x_ref, w_ref, out_ref, recv_ref, send_sems, recv_sems, credit_sems):
        my = lax.axis_index("i")
        left = (my - 1) % N_DEV
        right = (my + 1) % N_DEV

        barrier_sem = pltpu.get_barrier_semaphore()
        for nbr in (left, right):
            _sem_signal(barrier_sem, inc=1, device_id=(nbr,),
                        device_id_type=pl.DeviceIdType.MESH)
        _sem_wait(barrier_sem, 2)

        for s in (0, 1):
            _sem_signal(credit_sems.at[s], inc=1, device_id=(left,),
                        device_id_type=pl.DeviceIdType.MESH)

        w_bf16 = w_ref[...].astype(jnp.bfloat16)

        def gemm_body(i, _):
            xs = x_ref[pl.ds(i * 256, 256), :].astype(jnp.bfloat16)
            out_ref[pl.ds(i * 256, 256), :] = lax.dot_general(
                xs, w_bf16,
                dimension_numbers=(((1,), (0,)), ((), ())),
                preferred_element_type=jnp.float32,
            )
            return 0

        lax.fori_loop(0, m // 256, gemm_body, 0)

        def hop(g, sa):
            slot = lax.rem(g, 2)
            _sem_wait(credit_sems.at[slot], 1)
            rdma = pltpu.make_async_remote_copy(
                src_ref=out_ref.at[pl.ds(sa * ch, ch), :],
                dst_ref=recv_ref.at[slot],
                send_sem=send_sems.at[slot],
                recv_sem=recv_sems.at[slot],
                device_id=(right,),
                device_id_type=pl.DeviceIdType.MESH,
            )
            rdma.start()
            rdma.wait()
            return slot

        def rs_body(s, _):
            sa = (my - s) % N_DEV
            ra = (my - s - 1) % N_DEV
            slot = hop(s, sa)
            out_ref[pl.ds(ra * ch, ch), :] += recv_ref[slot]
            _sem_signal(credit_sems.at[slot], inc=1, device_id=(left,),
                        device_id_type=pl.DeviceIdType.MESH)
            return 0

        lax.fori_loop(0, N_DEV - 1, rs_body, 0)

        def ag_body(t, _):
            g = t + (N_DEV - 1)
            sa = (my + 1 - t) % N_DEV
            ra = (my - t) % N_DEV
            slot = hop(g, sa)
            out_ref[pl.ds(ra * ch, ch), :] = recv_ref[slot]
            _sem_signal(credit_sems.at[slot], inc=1, device_id=(left,),
                        device_id_type=pl.DeviceIdType.MESH)
            return 0

        lax.fori_loop(0, N_DEV - 1, ag_body, 0)

        _sem_wait(credit_sems.at[0], 1)
        _sem_wait(credit_sems.at[1], 1)

        def amax_body(i, acc):
            c = out_ref[pl.ds(i * 256, 256), :]
            return jnp.maximum(acc, jnp.max(c))

        amax = lax.fori_loop(0, m // 256, amax_body, jnp.float32(0.0))
        scale = amax / 127.0

        def quant_body(i, _):
            y = jnp.maximum(out_ref[pl.ds(i * 256, 256), :], 0.0)
            q = jnp.clip(jnp.round(y / scale), -127.0, 127.0)
            out_ref[pl.ds(i * 256, 256), :] = q * scale
            return 0

        lax.fori_loop(0, m // 256, quant_body, 0)

    return pl.pallas_call(
        body,
        out_shape=jax.ShapeDtypeStruct((m, n), jnp.float32),
        in_specs=[
            pl.BlockSpec(memory_space=pltpu.VMEM),
            pl.BlockSpec(memory_space=pltpu.VMEM),
        ],
        out_specs=pl.BlockSpec(memory_space=pltpu.VMEM),
        scratch_shapes=[
            pltpu.VMEM((2, ch, n), jnp.float32),
            pltpu.SemaphoreType.DMA((2,)),
            pltpu.SemaphoreType.DMA((2,)),
            pltpu.SemaphoreType.REGULAR((2,)),
        ],
        compiler_params=_CompilerParams(
            collective_id=0, vmem_limit_bytes=56 * 1024 * 1024
        ),
    )(x, w_mat)


# device time: 537341 ns/iter; 1.6415x vs baseline; 1.6415x over previous
import jax
import jax.numpy as jnp
from jax import lax
from jax.experimental import pallas as pl
from jax.experimental.pallas import tpu as pltpu

N_DEV = 32

_sem_signal = getattr(pl, "semaphore_signal", None) or pltpu.semaphore_signal
_sem_wait = getattr(pl, "semaphore_wait", None) or pltpu.semaphore_wait
_CompilerParams = getattr(pltpu, "CompilerParams", None) or pltpu.TPUCompilerParams


def kernel(x, w_mat):
    m, _ = x.shape
    _, n = w_mat.shape
    ch = m // N_DEV

    def body(x_ref, w_ref, out_ref, send_ref, recv_ref, send_sems, recv_sems,
             credit_sems):
        my = lax.axis_index("i")
        left = (my - 1) % N_DEV
        right = (my + 1) % N_DEV

        barrier_sem = pltpu.get_barrier_semaphore()
        for nbr in (left, right):
            _sem_signal(barrier_sem, inc=1, device_id=(nbr,),
                        device_id_type=pl.DeviceIdType.MESH)
        _sem_wait(barrier_sem, 2)

        for s in (0, 1):
            _sem_signal(credit_sems.at[s], inc=1, device_id=(left,),
                        device_id_type=pl.DeviceIdType.MESH)

        w_bf16 = w_ref[...].astype(jnp.bfloat16)

        def gemm_body(i, _):
            xs = x_ref[pl.ds(i * 256, 256), :].astype(jnp.bfloat16)
            out_ref[pl.ds(i * 256, 256), :] = lax.dot_general(
                xs, w_bf16,
                dimension_numbers=(((1,), (0,)), ((), ())),
                preferred_element_type=jnp.float32,
            )
            return 0

        lax.fori_loop(0, m // 256, gemm_body, 0)

        def hop(g, sa):
            slot = lax.rem(g, 2)
            _sem_wait(credit_sems.at[slot], 1)
            send_ref[slot] = out_ref[pl.ds(sa * ch, ch), :].astype(jnp.bfloat16)
            rdma = pltpu.make_async_remote_copy(
                src_ref=send_ref.at[slot],
                dst_ref=recv_ref.at[slot],
                send_sem=send_sems.at[slot],
                recv_sem=recv_sems.at[slot],
                device_id=(right,),
                device_id_type=pl.DeviceIdType.MESH,
            )
            rdma.start()
            rdma.wait()
            return slot

        def rs_body(s, _):
            sa = (my - s) % N_DEV
            ra = (my - s - 1) % N_DEV
            slot = hop(s, sa)
            out_ref[pl.ds(ra * ch, ch), :] += recv_ref[slot].astype(jnp.float32)
            _sem_signal(credit_sems.at[slot], inc=1, device_id=(left,),
                        device_id_type=pl.DeviceIdType.MESH)
            return 0

        lax.fori_loop(0, N_DEV - 1, rs_body, 0)

        def ag_body(t, _):
            g = t + (N_DEV - 1)
            sa = (my + 1 - t) % N_DEV
            ra = (my - t) % N_DEV
            slot = hop(g, sa)
            out_ref[pl.ds(ra * ch, ch), :] = recv_ref[slot].astype(jnp.float32)
            _sem_signal(credit_sems.at[slot], inc=1, device_id=(left,),
                        device_id_type=pl.DeviceIdType.MESH)
            return 0

        lax.fori_loop(0, N_DEV - 1, ag_body, 0)

        _sem_wait(credit_sems.at[0], 1)
        _sem_wait(credit_sems.at[1], 1)

        def amax_body(i, acc):
            c = out_ref[pl.ds(i * 256, 256), :]
            return jnp.maximum(acc, jnp.max(c))

        amax = lax.fori_loop(0, m // 256, amax_body, jnp.float32(0.0))
        scale = amax / 127.0

        def quant_body(i, _):
            y = jnp.maximum(out_ref[pl.ds(i * 256, 256), :], 0.0)
            q = jnp.clip(jnp.round(y / scale), -127.0, 127.0)
            out_ref[pl.ds(i * 256, 256), :] = q * scale
            return 0

        lax.fori_loop(0, m // 256, quant_body, 0)

    return pl.pallas_call(
        body,
        out_shape=jax.ShapeDtypeStruct((m, n), jnp.float32),
        in_specs=[
            pl.BlockSpec(memory_space=pltpu.VMEM),
            pl.BlockSpec(memory_space=pltpu.VMEM),
        ],
        out_specs=pl.BlockSpec(memory_space=pltpu.VMEM),
        scratch_shapes=[
            pltpu.VMEM((2, ch, n), jnp.bfloat16),
            pltpu.VMEM((2, ch, n), jnp.bfloat16),
            pltpu.SemaphoreType.DMA((2,)),
            pltpu.SemaphoreType.DMA((2,)),
            pltpu.SemaphoreType.REGULAR((2,)),
        ],
        compiler_params=_CompilerParams(
            collective_id=0, vmem_limit_bytes=56 * 1024 * 1024
        ),
    )(x, w_mat)
